# baseline (speedup 1.0000x reference)
import functools

import jax
import jax.numpy as jnp
from jax import lax
from jax.experimental import pallas as pl
from jax.experimental.pallas import tpu as pltpu
from jax.experimental.pallas import tpu_sc as plsc

KHOPS = 3
NNODES = 10000
NPAD = 10112
NEDGES = 320000
NSUB = 16
NCORE = 2
CHUNK = 128
EPT = 20480
NCHUNK = EPT // CHUNK
EPAD = EPT * NSUB
FH = 64
RB = 16
NRCHUNK = NNODES // RB
RC_PT = 40
ZROWS = NPAD // NSUB
F32 = jnp.float32

_MESH = plsc.VectorSubcoreMesh(core_axis_name="core", subcore_axis_name="subcore")
_SC_PARAMS = pltpu.CompilerParams(use_tc_tiling_on_sc=False)


def _sc_deg_body(dst_hbm, ones_hbm, zeros_hbm, deg_hbm, acc, dstv, onesb):
    c = lax.axis_index("core")
    s = lax.axis_index("subcore")
    pltpu.sync_copy(dst_hbm.at[s], dstv)
    pltpu.sync_copy(ones_hbm, onesb)
    pltpu.sync_copy(zeros_hbm, acc.at[pl.ds(s * ZROWS, ZROWS)])
    plsc.subcore_barrier()

    @pl.loop(0, NCHUNK)
    def _(j):
        pltpu.sync_copy(onesb, acc.at[dstv.at[j]], add=True)

    plsc.subcore_barrier()

    @pl.when(c == 0)
    def _():
        pltpu.sync_copy(acc.at[pl.ds(s * ZROWS, ZROWS)],
                        deg_hbm.at[pl.ds(s * ZROWS, ZROWS)])


def _sc_deg(dstp, ones16, zeros16):
    fn = pl.kernel(
        _sc_deg_body,
        out_type=jax.ShapeDtypeStruct((NPAD, 16), F32),
        mesh=_MESH,
        scratch_types=[
            pltpu.VMEM_SHARED((NPAD, 16), F32),
            pltpu.VMEM((NCHUNK, CHUNK), jnp.int32),
            pltpu.VMEM((CHUNK, 16), F32),
        ],
        compiler_params=_SC_PARAMS,
    )
    return fn(dstp, ones16, zeros16)


NBUF = 2


def _sc_prop_body(u0_hbm, src_hbm, dst_hbm, dinv_hbm, zeros_hbm,
                  h_hbm, u_hbm, acc, srcv, dstv, gb0, gb1, gb2, gb3, dinvv,
                  sbuf, hbuf, ubuf, sm0, sm1, sm2, sm3):
    c = lax.axis_index("core")
    s = lax.axis_index("subcore")
    gbufs = (gb0, gb1, gb2, gb3)[:NBUF]
    gsems = (sm0, sm1, sm2, sm3)[:NBUF]
    pltpu.sync_copy(src_hbm.at[s], srcv)
    pltpu.sync_copy(dst_hbm.at[s], dstv)
    pltpu.sync_copy(dinv_hbm, dinvv)
    lo = s * RC_PT
    hi = lax.min(jnp.int32(NRCHUNK), lo + RC_PT)

    for k in range(KHOPS):
        gsrc = u0_hbm if k == 0 else u_hbm
        gview = gsrc.at[c]

        def wait_g(t):
            pltpu.make_async_copy(gview.at[pl.ds(0, CHUNK)],
                                  gbufs[t], gsems[t]).wait()

        pltpu.sync_copy(zeros_hbm, acc.at[pl.ds(s * ZROWS, ZROWS)])
        plsc.subcore_barrier()

        for t in range(NBUF):
            pltpu.async_copy(gview.at[srcv.at[t]], gbufs[t], gsems[t])

        @pl.loop(0, NCHUNK - NBUF, step=NBUF)
        def _(j):
            for t in range(NBUF):
                wait_g(t)
                pltpu.sync_copy(gbufs[t], acc.at[dstv.at[j + t]], add=True)
                pltpu.async_copy(gview.at[srcv.at[j + NBUF + t]],
                                 gbufs[t], gsems[t])

        for t in range(NBUF):
            wait_g(t)
            pltpu.sync_copy(gbufs[t], acc.at[dstv.at[NCHUNK - NBUF + t]],
                            add=True)

        plsc.subcore_barrier()

        @pl.loop(lo, hi)
        def _(i):
            r0 = i * RB
            pltpu.sync_copy(acc.at[pl.ds(r0, RB)], sbuf)
            dvec = dinvv[pl.ds(r0, RB)]
            for j in range(RB):
                d = dvec[j]
                for q in range(FH // 16):
                    v = sbuf[j, pl.ds(q * 16, 16)]
                    hv = v * d
                    hbuf[j, pl.ds(q * 16, 16)] = hv
                    ubuf[j, pl.ds(q * 16, 16)] = hv * d
            pltpu.sync_copy(hbuf, h_hbm.at[k].at[c].at[pl.ds(r0, RB)])
            pltpu.sync_copy(ubuf, u_hbm.at[c].at[pl.ds(r0, RB)])

        plsc.subcore_barrier()


def _sc_prop(u0, srcp, dstp, dinv_flat, zeros64):
    fn = pl.kernel(
        _sc_prop_body,
        out_type=[
            jax.ShapeDtypeStruct((KHOPS, NCORE, NNODES, FH), F32),
            jax.ShapeDtypeStruct((NCORE, NNODES, FH), F32),
        ],
        mesh=_MESH,
        scratch_types=[
            pltpu.VMEM_SHARED((NPAD, FH), F32),
            pltpu.VMEM((NCHUNK, CHUNK), jnp.int32),
            pltpu.VMEM((NCHUNK, CHUNK), jnp.int32),
            pltpu.VMEM((CHUNK, FH), F32),
            pltpu.VMEM((CHUNK, FH), F32),
            pltpu.VMEM((CHUNK, FH), F32),
            pltpu.VMEM((CHUNK, FH), F32),
            pltpu.VMEM((NNODES,), F32),
            pltpu.VMEM((RB, FH), F32),
            pltpu.VMEM((RB, FH), F32),
            pltpu.VMEM((RB, FH), F32),
            pltpu.SemaphoreType.DMA,
            pltpu.SemaphoreType.DMA,
            pltpu.SemaphoreType.DMA,
            pltpu.SemaphoreType.DMA,
        ],
        compiler_params=_SC_PARAMS,
    )
    return fn(u0, srcp, dstp, dinv_flat, zeros64)


_RT = 1000


def _tc_prep_body(deg_ref, x_ref, dinv_ref, u0_ref):
    deg = deg_ref[...][:, 0:1]
    d = jnp.where(deg > 0, lax.rsqrt(jnp.maximum(deg, 1e-12)), 0.0)
    dinv_ref[...] = d
    u = x_ref[...] * d
    u0_ref[0] = u[:, :FH]
    u0_ref[1] = u[:, FH:]


def _tc_prep(deg16, x):
    return pl.pallas_call(
        _tc_prep_body,
        grid=(NNODES // _RT,),
        in_specs=[
            pl.BlockSpec((_RT, 16), lambda i: (i, 0)),
            pl.BlockSpec((_RT, 128), lambda i: (i, 0)),
        ],
        out_specs=[
            pl.BlockSpec((_RT, 1), lambda i: (i, 0)),
            pl.BlockSpec((NCORE, _RT, FH), lambda i: (0, i, 0)),
        ],
        out_shape=[
            jax.ShapeDtypeStruct((NNODES, 1), F32),
            jax.ShapeDtypeStruct((NCORE, NNODES, FH), F32),
        ],
    )(deg16, x)


def _dot(a, b):
    return jnp.dot(a, b, preferred_element_type=F32,
                   precision=lax.Precision.HIGHEST)


def _tc_layer1_body(x_ref, h_ref, w_ref, b_ref, dinv_ref, x1_ref, u_ref):
    acc = _dot(x_ref[...], w_ref[0])
    for k in range(KHOPS):
        hk = jnp.concatenate([h_ref[k, 0], h_ref[k, 1]], axis=1)
        acc = acc + _dot(hk, w_ref[k + 1])
    acc = jnp.maximum(acc + b_ref[...], 0.0)
    x1_ref[...] = acc
    u = acc * dinv_ref[...]
    u_ref[0] = u[:, :FH]
    u_ref[1] = u[:, FH:]


def _tc_layer1(x, h1, W1, b1, dinv):
    return pl.pallas_call(
        _tc_layer1_body,
        grid=(NNODES // _RT,),
        in_specs=[
            pl.BlockSpec((_RT, 128), lambda i: (i, 0)),
            pl.BlockSpec((KHOPS, NCORE, _RT, FH), lambda i: (0, 0, i, 0)),
            pl.BlockSpec((KHOPS + 1, 128, 128), lambda i: (0, 0, 0)),
            pl.BlockSpec((1, 128), lambda i: (0, 0)),
            pl.BlockSpec((_RT, 1), lambda i: (i, 0)),
        ],
        out_specs=[
            pl.BlockSpec((_RT, 128), lambda i: (i, 0)),
            pl.BlockSpec((NCORE, _RT, FH), lambda i: (0, i, 0)),
        ],
        out_shape=[
            jax.ShapeDtypeStruct((NNODES, 128), F32),
            jax.ShapeDtypeStruct((NCORE, NNODES, FH), F32),
        ],
    )(x, h1, W1, b1.reshape(1, 128), dinv)


def _tc_layer2_body(x_ref, h_ref, w_ref, b_ref, out_ref):
    acc = _dot(x_ref[...], w_ref[0])
    for k in range(KHOPS):
        hk = jnp.concatenate([h_ref[k, 0], h_ref[k, 1]], axis=1)
        acc = acc + _dot(hk, w_ref[k + 1])
    acc = acc + b_ref[...]
    m = jnp.max(acc, axis=1, keepdims=True)
    lse = jnp.log(jnp.sum(jnp.exp(acc - m), axis=1, keepdims=True)) + m
    out_ref[...] = acc - lse


def _tc_layer2(x1, h2, W2, b2):
    cls = W2.shape[-1]
    return pl.pallas_call(
        _tc_layer2_body,
        grid=(NNODES // _RT,),
        in_specs=[
            pl.BlockSpec((_RT, 128), lambda i: (i, 0)),
            pl.BlockSpec((KHOPS, NCORE, _RT, FH), lambda i: (0, 0, i, 0)),
            pl.BlockSpec((KHOPS + 1, 128, cls), lambda i: (0, 0, 0)),
            pl.BlockSpec((1, cls), lambda i: (0, 0)),
        ],
        out_specs=pl.BlockSpec((_RT, cls), lambda i: (i, 0)),
        out_shape=jax.ShapeDtypeStruct((NNODES, cls), F32),
    )(x1, h2, W2, b2.reshape(1, cls))


def kernel(x, edge_index, W1, b1, W2, b2):
    src = edge_index[0]
    dst = edge_index[1]
    pad = EPAD - NEDGES
    srcp = jnp.concatenate([src, jnp.zeros((pad,), jnp.int32)])
    dstp = jnp.concatenate([dst, jnp.full((pad,), NNODES, jnp.int32)])
    srcp = srcp.reshape(NSUB, NCHUNK, CHUNK)
    dstp = dstp.reshape(NSUB, NCHUNK, CHUNK)
    ones16 = jnp.ones((CHUNK, 16), F32)
    zeros16 = jnp.zeros((ZROWS, 16), F32)
    zeros64 = jnp.zeros((ZROWS, FH), F32)

    deg16 = _sc_deg(dstp, ones16, zeros16)
    dinv, u0 = _tc_prep(deg16, x)
    dinv_flat = dinv.reshape(NNODES)
    h1, _ = _sc_prop(u0, srcp, dstp, dinv_flat, zeros64)
    x1, u2 = _tc_layer1(x, h1, W1, b1, dinv)
    h2, _ = _sc_prop(u2, srcp, dstp, dinv_flat, zeros64)
    return _tc_layer2(x1, h2, W2, b2)


# speedup vs baseline: 1.0004x; 1.0004x over previous
import functools

import jax
import jax.numpy as jnp
from jax import lax
from jax.experimental import pallas as pl
from jax.experimental.pallas import tpu as pltpu
from jax.experimental.pallas import tpu_sc as plsc

KHOPS = 3
NNODES = 10000
NPAD = 10112
NEDGES = 320000
NSUB = 16
NCORE = 2
CHUNK = 128
EPT = 20480
NCHUNK = EPT // CHUNK
EPAD = EPT * NSUB
FH = 64
RB = 16
NRCHUNK = NNODES // RB
RC_PT = 40
ZROWS = NPAD // NSUB
F32 = jnp.float32

_MESH = plsc.VectorSubcoreMesh(core_axis_name="core", subcore_axis_name="subcore")
_SC_PARAMS = pltpu.CompilerParams(use_tc_tiling_on_sc=False)


def _sc_deg_body(dst_hbm, ones_hbm, zeros_hbm, deg_hbm, acc, dstv, onesb):
    c = lax.axis_index("core")
    s = lax.axis_index("subcore")
    pltpu.sync_copy(dst_hbm.at[s], dstv)
    pltpu.sync_copy(ones_hbm, onesb)
    pltpu.sync_copy(zeros_hbm, acc.at[pl.ds(s * ZROWS, ZROWS)])
    plsc.subcore_barrier()

    @pl.loop(0, NCHUNK)
    def _(j):
        pltpu.sync_copy(onesb, acc.at[dstv.at[j]], add=True)

    plsc.subcore_barrier()

    @pl.when(c == 0)
    def _():
        pltpu.sync_copy(acc.at[pl.ds(s * ZROWS, ZROWS)],
                        deg_hbm.at[pl.ds(s * ZROWS, ZROWS)])


def _sc_deg(dstp, ones16, zeros16):
    fn = pl.kernel(
        _sc_deg_body,
        out_type=jax.ShapeDtypeStruct((NPAD, 16), F32),
        mesh=_MESH,
        scratch_types=[
            pltpu.VMEM_SHARED((NPAD, 16), F32),
            pltpu.VMEM((NCHUNK, CHUNK), jnp.int32),
            pltpu.VMEM((CHUNK, 16), F32),
        ],
        compiler_params=_SC_PARAMS,
    )
    return fn(dstp, ones16, zeros16)


NBUF = 2


def _sc_prop_body(u0_hbm, src_hbm, dst_hbm, dinv_hbm, zeros_hbm,
                  h_hbm, u_hbm, acc, srcv, dstv, gb0, gb1, gb2, gb3, dinvv,
                  sbuf, hbuf, ubuf, sm0, sm1, sm2, sm3):
    c = lax.axis_index("core")
    s = lax.axis_index("subcore")
    gbufs = (gb0, gb1, gb2, gb3)[:NBUF]
    gsems = (sm0, sm1, sm2, sm3)[:NBUF]
    pltpu.sync_copy(src_hbm.at[s], srcv)
    pltpu.sync_copy(dst_hbm.at[s], dstv)
    pltpu.sync_copy(dinv_hbm, dinvv)
    lo = s * RC_PT
    hi = lax.min(jnp.int32(NRCHUNK), lo + RC_PT)

    for k in range(KHOPS):
        gsrc = u0_hbm if k == 0 else u_hbm
        gview = gsrc.at[c]

        def wait_g(t):
            pltpu.make_async_copy(gview.at[pl.ds(0, CHUNK)],
                                  gbufs[t], gsems[t]).wait()

        pltpu.sync_copy(zeros_hbm, acc.at[pl.ds(s * ZROWS, ZROWS)])
        plsc.subcore_barrier()

        for t in range(NBUF):
            pltpu.async_copy(gview.at[srcv.at[t]], gbufs[t], gsems[t])

        @pl.loop(0, NCHUNK - NBUF, step=NBUF)
        def _(j):
            for t in range(NBUF):
                wait_g(t)
                pltpu.sync_copy(gbufs[t], acc.at[dstv.at[j + t]], add=True)
                pltpu.async_copy(gview.at[srcv.at[j + NBUF + t]],
                                 gbufs[t], gsems[t])

        for t in range(NBUF):
            wait_g(t)
            pltpu.sync_copy(gbufs[t], acc.at[dstv.at[NCHUNK - NBUF + t]],
                            add=True)

        plsc.subcore_barrier()

        @pl.loop(lo, hi)
        def _(i):
            r0 = i * RB
            pltpu.sync_copy(acc.at[pl.ds(r0, RB)], sbuf)
            dvec = dinvv[pl.ds(r0, RB)]
            for j in range(RB):
                d = dvec[j]
                for q in range(FH // 16):
                    v = sbuf[j, pl.ds(q * 16, 16)]
                    hv = v * d
                    hbuf[j, pl.ds(q * 16, 16)] = hv
                    ubuf[j, pl.ds(q * 16, 16)] = hv * d
            pltpu.sync_copy(hbuf, h_hbm.at[k].at[c].at[pl.ds(r0, RB)])
            pltpu.sync_copy(ubuf, u_hbm.at[c].at[pl.ds(r0, RB)])

        plsc.subcore_barrier()


def _sc_prop(u0, srcp, dstp, dinv_flat, zeros64):
    fn = pl.kernel(
        _sc_prop_body,
        out_type=[
            jax.ShapeDtypeStruct((KHOPS, NCORE, NNODES, FH), F32),
            jax.ShapeDtypeStruct((NCORE, NNODES, FH), F32),
        ],
        mesh=_MESH,
        scratch_types=[
            pltpu.VMEM_SHARED((NPAD, FH), F32),
            pltpu.VMEM((NCHUNK, CHUNK), jnp.int32),
            pltpu.VMEM((NCHUNK, CHUNK), jnp.int32),
            pltpu.VMEM((CHUNK, FH), F32),
            pltpu.VMEM((CHUNK, FH), F32),
            pltpu.VMEM((CHUNK, FH), F32),
            pltpu.VMEM((CHUNK, FH), F32),
            pltpu.VMEM((NNODES,), F32),
            pltpu.VMEM((RB, FH), F32),
            pltpu.VMEM((RB, FH), F32),
            pltpu.VMEM((RB, FH), F32),
            pltpu.SemaphoreType.DMA,
            pltpu.SemaphoreType.DMA,
            pltpu.SemaphoreType.DMA,
            pltpu.SemaphoreType.DMA,
        ],
        compiler_params=_SC_PARAMS,
    )
    return fn(u0, srcp, dstp, dinv_flat, zeros64)


_RT = 1000


def _tc_prep_body(deg_ref, x_ref, dinv_ref, u0_ref):
    deg = deg_ref[...][:, 0:1]
    d = jnp.where(deg > 0, lax.rsqrt(jnp.maximum(deg, 1e-12)), 0.0)
    dinv_ref[...] = d
    u = x_ref[...] * d
    u0_ref[0] = u[:, :FH]
    u0_ref[1] = u[:, FH:]


def _tc_prep(deg16, x):
    return pl.pallas_call(
        _tc_prep_body,
        grid=(NNODES // _RT,),
        in_specs=[
            pl.BlockSpec((_RT, 16), lambda i: (i, 0)),
            pl.BlockSpec((_RT, 128), lambda i: (i, 0)),
        ],
        out_specs=[
            pl.BlockSpec((_RT, 1), lambda i: (i, 0)),
            pl.BlockSpec((NCORE, _RT, FH), lambda i: (0, i, 0)),
        ],
        out_shape=[
            jax.ShapeDtypeStruct((NNODES, 1), F32),
            jax.ShapeDtypeStruct((NCORE, NNODES, FH), F32),
        ],
    )(deg16, x)


def _dot(a, b):
    return jnp.dot(a, b, preferred_element_type=F32,
                   precision=lax.Precision.HIGHEST)


def _tc_layer1_body(x_ref, h_ref, w_ref, b_ref, dinv_ref, x1_ref, u_ref):
    acc = _dot(x_ref[...], w_ref[0])
    for k in range(KHOPS):
        hk = jnp.concatenate([h_ref[k, 0], h_ref[k, 1]], axis=1)
        acc = acc + _dot(hk, w_ref[k + 1])
    acc = jnp.maximum(acc + b_ref[...], 0.0)
    x1_ref[...] = acc
    u = acc * dinv_ref[...]
    u_ref[0] = u[:, :FH]
    u_ref[1] = u[:, FH:]


def _tc_layer1(x, h1, W1, b1, dinv):
    return pl.pallas_call(
        _tc_layer1_body,
        grid=(NNODES // _RT,),
        in_specs=[
            pl.BlockSpec((_RT, 128), lambda i: (i, 0)),
            pl.BlockSpec((KHOPS, NCORE, _RT, FH), lambda i: (0, 0, i, 0)),
            pl.BlockSpec((KHOPS + 1, 128, 128), lambda i: (0, 0, 0)),
            pl.BlockSpec((1, 128), lambda i: (0, 0)),
            pl.BlockSpec((_RT, 1), lambda i: (i, 0)),
        ],
        out_specs=[
            pl.BlockSpec((_RT, 128), lambda i: (i, 0)),
            pl.BlockSpec((NCORE, _RT, FH), lambda i: (0, i, 0)),
        ],
        out_shape=[
            jax.ShapeDtypeStruct((NNODES, 128), F32),
            jax.ShapeDtypeStruct((NCORE, NNODES, FH), F32),
        ],
    )(x, h1, W1, b1.reshape(1, 128), dinv)


def _tc_layer2_body(x_ref, h_ref, w_ref, b_ref, out_ref):
    acc = _dot(x_ref[...], w_ref[0])
    for k in range(KHOPS):
        hk = jnp.concatenate([h_ref[k, 0], h_ref[k, 1]], axis=1)
        acc = acc + _dot(hk, w_ref[k + 1])
    acc = acc + b_ref[...]
    m = jnp.max(acc, axis=1, keepdims=True)
    lse = jnp.log(jnp.sum(jnp.exp(acc - m), axis=1, keepdims=True)) + m
    out_ref[...] = acc - lse


def _tc_layer2(x1, h2, W2, b2):
    cls = W2.shape[-1]
    return pl.pallas_call(
        _tc_layer2_body,
        grid=(NNODES // _RT,),
        in_specs=[
            pl.BlockSpec((_RT, 128), lambda i: (i, 0)),
            pl.BlockSpec((KHOPS, NCORE, _RT, FH), lambda i: (0, 0, i, 0)),
            pl.BlockSpec((KHOPS + 1, 128, cls), lambda i: (0, 0, 0)),
            pl.BlockSpec((1, cls), lambda i: (0, 0)),
        ],
        out_specs=pl.BlockSpec((_RT, cls), lambda i: (i, 0)),
        out_shape=jax.ShapeDtypeStruct((NNODES, cls), F32),
    )(x1, h2, W2, b2.reshape(1, cls))


def kernel(x, edge_index, W1, b1, W2, b2):
    src = edge_index[0]
    dst = edge_index[1]
    pad = EPAD - NEDGES
    trash = NNODES + jnp.arange(pad, dtype=jnp.int32) % (NPAD - NNODES)
    srcp = jnp.concatenate([src, jnp.zeros((pad,), jnp.int32)])
    dstp = jnp.concatenate([dst, trash])
    srcp = srcp.reshape(NSUB, NCHUNK, CHUNK)
    dstp = dstp.reshape(NSUB, NCHUNK, CHUNK)
    ones16 = jnp.ones((CHUNK, 16), F32)
    zeros16 = jnp.zeros((ZROWS, 16), F32)
    zeros64 = jnp.zeros((ZROWS, FH), F32)

    deg16 = _sc_deg(dstp, ones16, zeros16)
    dinv, u0 = _tc_prep(deg16, x)
    dinv_flat = dinv.reshape(NNODES)
    h1, _ = _sc_prop(u0, srcp, dstp, dinv_flat, zeros64)
    x1, u2 = _tc_layer1(x, h1, W1, b1, dinv)
    h2, _ = _sc_prop(u2, srcp, dstp, dinv_flat, zeros64)
    return _tc_layer2(x1, h2, W2, b2)


# speedup vs baseline: 1.4465x; 1.4459x over previous
import functools

import jax
import jax.numpy as jnp
from jax import lax
from jax.experimental import pallas as pl
from jax.experimental.pallas import tpu as pltpu
from jax.experimental.pallas import tpu_sc as plsc

KHOPS = 3
NNODES = 10000
NPAD = 10112
NEDGES = 320000
NSUB = 16
NCORE = 2
CHUNK = 128
EPT = 20224
NCHUNK = EPT // CHUNK
EPAD = EPT * NSUB
FH = 64
RB = 16
NRCHUNK = NNODES // RB
RC_PT = 40
ZROWS = NPAD // NSUB
F32 = jnp.float32

_MESH = plsc.VectorSubcoreMesh(core_axis_name="core", subcore_axis_name="subcore")
_SC_PARAMS = pltpu.CompilerParams(use_tc_tiling_on_sc=False)


def _sc_deg_body(dst_hbm, ones_hbm, zeros_hbm, deg_hbm, acc, dstv, onesb):
    c = lax.axis_index("core")
    s = lax.axis_index("subcore")
    pltpu.sync_copy(dst_hbm.at[s], dstv)
    pltpu.sync_copy(ones_hbm, onesb)
    pltpu.sync_copy(zeros_hbm, acc.at[pl.ds(s * ZROWS, ZROWS)])
    plsc.subcore_barrier()

    @pl.loop(0, NCHUNK)
    def _(j):
        pltpu.sync_copy(onesb, acc.at[dstv.at[j]], add=True)

    plsc.subcore_barrier()

    @pl.when(c == 0)
    def _():
        pltpu.sync_copy(acc.at[pl.ds(s * ZROWS, ZROWS)],
                        deg_hbm.at[pl.ds(s * ZROWS, ZROWS)])


def _sc_deg(dstp, ones16, zeros16):
    fn = pl.kernel(
        _sc_deg_body,
        out_type=jax.ShapeDtypeStruct((NPAD, 16), F32),
        mesh=_MESH,
        scratch_types=[
            pltpu.VMEM_SHARED((NPAD, 16), F32),
            pltpu.VMEM((NCHUNK, CHUNK), jnp.int32),
            pltpu.VMEM((CHUNK, 16), F32),
        ],
        compiler_params=_SC_PARAMS,
    )
    return fn(dstp, ones16, zeros16)


NBUF = 2


def _sc_prop_body(u0_hbm, src_hbm, dst_hbm, dinv_hbm, zeros_hbm,
                  h_hbm, u_hbm, acc, srcv, dstv, gb0, gb1, dinvv,
                  sbuf, hbuf, ubuf, sm0, sm1):
    c = lax.axis_index("core")
    s = lax.axis_index("subcore")
    gbufs = (gb0, gb1)
    gsems = (sm0, sm1)
    pltpu.sync_copy(src_hbm.at[s], srcv)
    pltpu.sync_copy(dst_hbm.at[s], dstv)
    pltpu.sync_copy(dinv_hbm, dinvv)
    lo = s * RC_PT
    hi = lax.min(jnp.int32(NRCHUNK), lo + RC_PT)

    for k in range(KHOPS):
        gsrc = u0_hbm if k == 0 else u_hbm
        gview = gsrc.at[c]

        def wait_g(t):
            pltpu.make_async_copy(gview.at[pl.ds(0, CHUNK)],
                                  gbufs[t], gsems[t]).wait()

        pltpu.sync_copy(zeros_hbm, acc.at[pl.ds(s * ZROWS, ZROWS)])
        plsc.subcore_barrier()

        for t in range(NBUF):
            pltpu.async_copy(gview.at[srcv.at[t]], gbufs[t], gsems[t])

        @pl.loop(0, NCHUNK - NBUF, step=NBUF)
        def _(j):
            for t in range(NBUF):
                wait_g(t)
                pltpu.sync_copy(gbufs[t], acc.at[dstv.at[j + t]], add=True)
                pltpu.async_copy(gview.at[srcv.at[j + NBUF + t]],
                                 gbufs[t], gsems[t])

        for t in range(NBUF):
            wait_g(t)
            pltpu.sync_copy(gbufs[t], acc.at[dstv.at[NCHUNK - NBUF + t]],
                            add=True)

        plsc.subcore_barrier()

        @pl.loop(lo, hi)
        def _(i):
            r0 = i * RB
            pltpu.sync_copy(acc.at[pl.ds(r0, RB)], sbuf)
            dvec = dinvv[pl.ds(r0, RB)]
            for j in range(RB):
                d = dvec[j]
                for q in range(FH // 16):
                    v = sbuf[j, pl.ds(q * 16, 16)]
                    hv = v * d
                    hbuf[j, pl.ds(q * 16, 16)] = hv
                    ubuf[j, pl.ds(q * 16, 16)] = hv * d
            pltpu.sync_copy(hbuf, h_hbm.at[k].at[c].at[pl.ds(r0, RB)])
            pltpu.sync_copy(ubuf, u_hbm.at[c].at[pl.ds(r0, RB)])

        plsc.subcore_barrier()


def _sc_prop(u0, srcp, dstp, dinv_flat, zeros64):
    fn = pl.kernel(
        _sc_prop_body,
        out_type=[
            jax.ShapeDtypeStruct((KHOPS, NCORE, NNODES, FH), F32),
            jax.ShapeDtypeStruct((NCORE, NNODES, FH), F32),
        ],
        mesh=_MESH,
        scratch_types=[
            pltpu.VMEM_SHARED((NPAD, FH), F32),
            pltpu.VMEM((NCHUNK, CHUNK), jnp.int32),
            pltpu.VMEM((NCHUNK, CHUNK), jnp.int32),
            pltpu.VMEM((CHUNK, FH), F32),
            pltpu.VMEM((CHUNK, FH), F32),
            pltpu.VMEM((NNODES,), F32),
            pltpu.VMEM((RB, FH), F32),
            pltpu.VMEM((RB, FH), F32),
            pltpu.VMEM((RB, FH), F32),
            pltpu.SemaphoreType.DMA,
            pltpu.SemaphoreType.DMA,
        ],
        compiler_params=_SC_PARAMS,
    )
    return fn(u0, srcp, dstp, dinv_flat, zeros64)


_RT = 1000


def _tc_prep_body(deg_ref, x_ref, dinv_ref, u0_ref):
    deg = deg_ref[...][:, 0:1]
    d = jnp.where(deg > 0, lax.rsqrt(jnp.maximum(deg, 1e-12)), 0.0)
    dinv_ref[...] = d
    u = x_ref[...] * d
    u0_ref[0] = u[:, :FH]
    u0_ref[1] = u[:, FH:]


def _tc_prep(deg16, x):
    return pl.pallas_call(
        _tc_prep_body,
        grid=(NNODES // _RT,),
        in_specs=[
            pl.BlockSpec((_RT, 16), lambda i: (i, 0)),
            pl.BlockSpec((_RT, 128), lambda i: (i, 0)),
        ],
        out_specs=[
            pl.BlockSpec((_RT, 1), lambda i: (i, 0)),
            pl.BlockSpec((NCORE, _RT, FH), lambda i: (0, i, 0)),
        ],
        out_shape=[
            jax.ShapeDtypeStruct((NNODES, 1), F32),
            jax.ShapeDtypeStruct((NCORE, NNODES, FH), F32),
        ],
    )(deg16, x)


def _dot(a, b):
    return jnp.dot(a, b, preferred_element_type=F32,
                   precision=lax.Precision.HIGHEST)


def _tc_layer1_body(x_ref, h_ref, w_ref, b_ref, dinv_ref, x1_ref, u_ref):
    acc = _dot(x_ref[...], w_ref[0])
    for k in range(KHOPS):
        hk = jnp.concatenate([h_ref[k, 0], h_ref[k, 1]], axis=1)
        acc = acc + _dot(hk, w_ref[k + 1])
    acc = jnp.maximum(acc + b_ref[...], 0.0)
    x1_ref[...] = acc
    u = acc * dinv_ref[...]
    u_ref[0] = u[:, :FH]
    u_ref[1] = u[:, FH:]


def _tc_layer1(x, h1, W1, b1, dinv):
    return pl.pallas_call(
        _tc_layer1_body,
        grid=(NNODES // _RT,),
        in_specs=[
            pl.BlockSpec((_RT, 128), lambda i: (i, 0)),
            pl.BlockSpec((KHOPS, NCORE, _RT, FH), lambda i: (0, 0, i, 0)),
            pl.BlockSpec((KHOPS + 1, 128, 128), lambda i: (0, 0, 0)),
            pl.BlockSpec((1, 128), lambda i: (0, 0)),
            pl.BlockSpec((_RT, 1), lambda i: (i, 0)),
        ],
        out_specs=[
            pl.BlockSpec((_RT, 128), lambda i: (i, 0)),
            pl.BlockSpec((NCORE, _RT, FH), lambda i: (0, i, 0)),
        ],
        out_shape=[
            jax.ShapeDtypeStruct((NNODES, 128), F32),
            jax.ShapeDtypeStruct((NCORE, NNODES, FH), F32),
        ],
    )(x, h1, W1, b1.reshape(1, 128), dinv)


def _tc_layer2_body(x_ref, h_ref, w_ref, b_ref, out_ref):
    acc = _dot(x_ref[...], w_ref[0])
    for k in range(KHOPS):
        hk = jnp.concatenate([h_ref[k, 0], h_ref[k, 1]], axis=1)
        acc = acc + _dot(hk, w_ref[k + 1])
    acc = acc + b_ref[...]
    m = jnp.max(acc, axis=1, keepdims=True)
    lse = jnp.log(jnp.sum(jnp.exp(acc - m), axis=1, keepdims=True)) + m
    out_ref[...] = acc - lse


def _tc_layer2(x1, h2, W2, b2):
    cls = W2.shape[-1]
    return pl.pallas_call(
        _tc_layer2_body,
        grid=(NNODES // _RT,),
        in_specs=[
            pl.BlockSpec((_RT, 128), lambda i: (i, 0)),
            pl.BlockSpec((KHOPS, NCORE, _RT, FH), lambda i: (0, 0, i, 0)),
            pl.BlockSpec((KHOPS + 1, 128, cls), lambda i: (0, 0, 0)),
            pl.BlockSpec((1, cls), lambda i: (0, 0)),
        ],
        out_specs=pl.BlockSpec((_RT, cls), lambda i: (i, 0)),
        out_shape=jax.ShapeDtypeStruct((NNODES, cls), F32),
    )(x1, h2, W2, b2.reshape(1, cls))


def kernel(x, edge_index, W1, b1, W2, b2):
    src = edge_index[0]
    dst = edge_index[1]
    pad = EPAD - NEDGES
    trash = NNODES + jnp.arange(pad, dtype=jnp.int32) % (NPAD - NNODES)
    srcp = jnp.concatenate([src, jnp.zeros((pad,), jnp.int32)])
    dstp = jnp.concatenate([dst, trash])
    srcp = srcp.reshape(NSUB, NCHUNK, CHUNK)
    dstp = dstp.reshape(NSUB, NCHUNK, CHUNK)
    ones16 = jnp.ones((CHUNK, 16), F32)
    zeros16 = jnp.zeros((ZROWS, 16), F32)
    zeros64 = jnp.zeros((ZROWS, FH), F32)

    deg16 = _sc_deg(dstp, ones16, zeros16)
    dinv, u0 = _tc_prep(deg16, x)
    dinv_flat = dinv.reshape(NNODES)
    h1, _ = _sc_prop(u0, srcp, dstp, dinv_flat, zeros64)
    x1, u2 = _tc_layer1(x, h1, W1, b1, dinv)
    h2, _ = _sc_prop(u2, srcp, dstp, dinv_flat, zeros64)
    return _tc_layer2(x1, h2, W2, b2)


# speedup vs baseline: 1.4832x; 1.0254x over previous
import functools

import jax
import jax.numpy as jnp
from jax import lax
from jax.experimental import pallas as pl
from jax.experimental.pallas import tpu as pltpu
from jax.experimental.pallas import tpu_sc as plsc

KHOPS = 3
NNODES = 10000
NPAD = 10112
NEDGES = 320000
NSUB = 16
NCORE = 2
CHUNK = 128
EPT = 20224
NCHUNK = EPT // CHUNK
EPAD = EPT * NSUB
FH = 64
RB = 16
NRCHUNK = NNODES // RB
RC_PT = 40
ZROWS = NPAD // NSUB
F32 = jnp.float32

_MESH = plsc.VectorSubcoreMesh(core_axis_name="core", subcore_axis_name="subcore")
_SC_PARAMS = pltpu.CompilerParams(use_tc_tiling_on_sc=False)


def _sc_deg_body(dst_hbm, ones_hbm, zeros_hbm, deg_hbm, acc, dstv, onesb):
    c = lax.axis_index("core")
    s = lax.axis_index("subcore")
    pltpu.sync_copy(dst_hbm.at[s], dstv)
    pltpu.sync_copy(ones_hbm, onesb)
    pltpu.sync_copy(zeros_hbm, acc.at[pl.ds(s * ZROWS, ZROWS)])
    plsc.subcore_barrier()

    @pl.loop(0, NCHUNK)
    def _(j):
        pltpu.sync_copy(onesb, acc.at[dstv.at[j]], add=True)

    plsc.subcore_barrier()

    @pl.when(c == 0)
    def _():
        pltpu.sync_copy(acc.at[pl.ds(s * ZROWS, ZROWS)],
                        deg_hbm.at[pl.ds(s * ZROWS, ZROWS)])


def _sc_deg(dstp, ones16, zeros16):
    fn = pl.kernel(
        _sc_deg_body,
        out_type=jax.ShapeDtypeStruct((NPAD, 16), F32),
        mesh=_MESH,
        scratch_types=[
            pltpu.VMEM_SHARED((NPAD, 16), F32),
            pltpu.VMEM((NCHUNK, CHUNK), jnp.int32),
            pltpu.VMEM((CHUNK, 16), F32),
        ],
        compiler_params=_SC_PARAMS,
    )
    return fn(dstp, ones16, zeros16)


NBUF = 2


def _sc_prop_body(u0_hbm, edge_hbm, dinv_hbm, zeros_hbm,
                  h_hbm, acc, usp, ib0, ib1, gb0, gb1, dinvv,
                  sbuf, hbuf, ubuf, sm0, sm1):
    c = lax.axis_index("core")
    s = lax.axis_index("subcore")
    gbufs = (gb0, gb1)
    ibufs = (ib0, ib1)
    gsems = (sm0, sm1)
    eview = edge_hbm.at[s]
    pltpu.sync_copy(dinv_hbm.at[pl.ds(s * (RC_PT * RB), RC_PT * RB)], dinvv)
    pltpu.sync_copy(u0_hbm.at[c].at[pl.ds(s * ZROWS, ZROWS)],
                    usp.at[pl.ds(s * ZROWS, ZROWS)])
    lo = s * RC_PT
    hi = lax.min(jnp.int32(NRCHUNK), lo + RC_PT)

    def wait_g(t):
        pltpu.make_async_copy(usp.at[pl.ds(0, CHUNK)],
                              gbufs[t], gsems[t]).wait()

    for k in range(KHOPS):
        pltpu.sync_copy(zeros_hbm, acc.at[pl.ds(s * ZROWS, ZROWS)])
        plsc.subcore_barrier()

        for t in range(NBUF):
            pltpu.sync_copy(eview.at[t], ibufs[t])
            pltpu.async_copy(usp.at[ibufs[t].at[0]], gbufs[t], gsems[t])

        @pl.loop(0, NCHUNK - NBUF, step=NBUF)
        def _(j):
            for t in range(NBUF):
                wait_g(t)
                pltpu.sync_copy(gbufs[t], acc.at[ibufs[t].at[1]], add=True)
                pltpu.sync_copy(eview.at[j + NBUF + t], ibufs[t])
                pltpu.async_copy(usp.at[ibufs[t].at[0]], gbufs[t], gsems[t])

        for t in range(NBUF):
            wait_g(t)
            pltpu.sync_copy(gbufs[t], acc.at[ibufs[t].at[1]], add=True)

        plsc.subcore_barrier()

        @pl.loop(lo, hi)
        def _(i):
            r0 = i * RB
            pltpu.sync_copy(acc.at[pl.ds(r0, RB)], sbuf)
            dvec = dinvv[pl.ds((i - lo) * RB, RB)]
            for j in range(RB):
                d = dvec[j]
                for q in range(FH // 16):
                    v = sbuf[j, pl.ds(q * 16, 16)]
                    hv = v * d
                    hbuf[j, pl.ds(q * 16, 16)] = hv
                    ubuf[j, pl.ds(q * 16, 16)] = hv * d
            pltpu.sync_copy(hbuf, h_hbm.at[k].at[c].at[pl.ds(r0, RB)])
            pltpu.sync_copy(ubuf, usp.at[pl.ds(r0, RB)])

        plsc.subcore_barrier()


def _sc_prop(u0, edges, dinv_pad, zeros64):
    fn = pl.kernel(
        _sc_prop_body,
        out_type=jax.ShapeDtypeStruct((KHOPS, NCORE, NNODES, FH), F32),
        mesh=_MESH,
        scratch_types=[
            pltpu.VMEM_SHARED((NPAD, FH), F32),
            pltpu.VMEM_SHARED((NPAD, FH), F32),
            pltpu.VMEM((2, CHUNK), jnp.int32),
            pltpu.VMEM((2, CHUNK), jnp.int32),
            pltpu.VMEM((CHUNK, FH), F32),
            pltpu.VMEM((CHUNK, FH), F32),
            pltpu.VMEM((RC_PT * RB,), F32),
            pltpu.VMEM((RB, FH), F32),
            pltpu.VMEM((RB, FH), F32),
            pltpu.VMEM((RB, FH), F32),
            pltpu.SemaphoreType.DMA,
            pltpu.SemaphoreType.DMA,
        ],
        compiler_params=_SC_PARAMS,
    )
    return fn(u0, edges, dinv_pad, zeros64)


_RT = 1000


def _tc_prep_body(deg_ref, x_ref, dinv_ref, u0_ref):
    deg = deg_ref[...][:, 0:1]
    d = jnp.where(deg > 0, lax.rsqrt(jnp.maximum(deg, 1e-12)), 0.0)
    dinv_ref[...] = d
    u = x_ref[...] * d
    u0_ref[0] = u[:, :FH]
    u0_ref[1] = u[:, FH:]


def _tc_prep(deg16, x):
    return pl.pallas_call(
        _tc_prep_body,
        grid=(NNODES // _RT,),
        in_specs=[
            pl.BlockSpec((_RT, 16), lambda i: (i, 0)),
            pl.BlockSpec((_RT, 128), lambda i: (i, 0)),
        ],
        out_specs=[
            pl.BlockSpec((_RT, 1), lambda i: (i, 0)),
            pl.BlockSpec((NCORE, _RT, FH), lambda i: (0, i, 0)),
        ],
        out_shape=[
            jax.ShapeDtypeStruct((NNODES, 1), F32),
            jax.ShapeDtypeStruct((NCORE, NPAD, FH), F32),
        ],
    )(deg16, x)


def _dot(a, b):
    return jnp.dot(a, b, preferred_element_type=F32,
                   precision=lax.Precision.HIGHEST)


def _tc_layer1_body(x_ref, h_ref, w_ref, b_ref, dinv_ref, x1_ref, u_ref):
    acc = _dot(x_ref[...], w_ref[0])
    for k in range(KHOPS):
        hk = jnp.concatenate([h_ref[k, 0], h_ref[k, 1]], axis=1)
        acc = acc + _dot(hk, w_ref[k + 1])
    acc = jnp.maximum(acc + b_ref[...], 0.0)
    x1_ref[...] = acc
    u = acc * dinv_ref[...]
    u_ref[0] = u[:, :FH]
    u_ref[1] = u[:, FH:]


def _tc_layer1(x, h1, W1, b1, dinv):
    return pl.pallas_call(
        _tc_layer1_body,
        grid=(NNODES // _RT,),
        in_specs=[
            pl.BlockSpec((_RT, 128), lambda i: (i, 0)),
            pl.BlockSpec((KHOPS, NCORE, _RT, FH), lambda i: (0, 0, i, 0)),
            pl.BlockSpec((KHOPS + 1, 128, 128), lambda i: (0, 0, 0)),
            pl.BlockSpec((1, 128), lambda i: (0, 0)),
            pl.BlockSpec((_RT, 1), lambda i: (i, 0)),
        ],
        out_specs=[
            pl.BlockSpec((_RT, 128), lambda i: (i, 0)),
            pl.BlockSpec((NCORE, _RT, FH), lambda i: (0, i, 0)),
        ],
        out_shape=[
            jax.ShapeDtypeStruct((NNODES, 128), F32),
            jax.ShapeDtypeStruct((NCORE, NPAD, FH), F32),
        ],
    )(x, h1, W1, b1.reshape(1, 128), dinv)


def _tc_layer2_body(x_ref, h_ref, w_ref, b_ref, out_ref):
    acc = _dot(x_ref[...], w_ref[0])
    for k in range(KHOPS):
        hk = jnp.concatenate([h_ref[k, 0], h_ref[k, 1]], axis=1)
        acc = acc + _dot(hk, w_ref[k + 1])
    acc = acc + b_ref[...]
    m = jnp.max(acc, axis=1, keepdims=True)
    lse = jnp.log(jnp.sum(jnp.exp(acc - m), axis=1, keepdims=True)) + m
    out_ref[...] = acc - lse


def _tc_layer2(x1, h2, W2, b2):
    cls = W2.shape[-1]
    return pl.pallas_call(
        _tc_layer2_body,
        grid=(NNODES // _RT,),
        in_specs=[
            pl.BlockSpec((_RT, 128), lambda i: (i, 0)),
            pl.BlockSpec((KHOPS, NCORE, _RT, FH), lambda i: (0, 0, i, 0)),
            pl.BlockSpec((KHOPS + 1, 128, cls), lambda i: (0, 0, 0)),
            pl.BlockSpec((1, cls), lambda i: (0, 0)),
        ],
        out_specs=pl.BlockSpec((_RT, cls), lambda i: (i, 0)),
        out_shape=jax.ShapeDtypeStruct((NNODES, cls), F32),
    )(x1, h2, W2, b2.reshape(1, cls))


def kernel(x, edge_index, W1, b1, W2, b2):
    src = edge_index[0]
    dst = edge_index[1]
    pad = EPAD - NEDGES
    trash = NNODES + jnp.arange(pad, dtype=jnp.int32) % (NPAD - NNODES)
    srcp = jnp.concatenate([src, jnp.zeros((pad,), jnp.int32)])
    dstp = jnp.concatenate([dst, trash])
    srcp = srcp.reshape(NSUB, NCHUNK, CHUNK)
    dstp = dstp.reshape(NSUB, NCHUNK, CHUNK)
    ones16 = jnp.ones((CHUNK, 16), F32)
    zeros16 = jnp.zeros((ZROWS, 16), F32)
    zeros64 = jnp.zeros((ZROWS, FH), F32)

    edges = jnp.stack([srcp, dstp], axis=2)
    deg16 = _sc_deg(dstp, ones16, zeros16)
    dinv, u0 = _tc_prep(deg16, x)
    dinv_pad = jnp.concatenate(
        [dinv.reshape(NNODES),
         jnp.zeros((NSUB * RC_PT * RB - NNODES,), F32)])
    h1 = _sc_prop(u0, edges, dinv_pad, zeros64)
    x1, u2 = _tc_layer1(x, h1, W1, b1, dinv)
    h2 = _sc_prop(u2, edges, dinv_pad, zeros64)
    return _tc_layer2(x1, h2, W2, b2)
